# Initial kernel scaffold; baseline (speedup 1.0000x reference)
import jax
import jax.numpy as jnp
from jax.experimental import pallas as pl


def kernel(x, resids, mask, table):
    raise NotImplementedError("write your pallas kernel here")


# speedup vs baseline: 2.8749x; 2.8749x over previous
import jax
import jax.numpy as jnp
from jax.experimental import pallas as pl


_BLOCK = 2048


def _body(x_ref, r_ref, m_ref, t_ref, o_ref):
    r = r_ref[...]
    m = m_ref[...]
    xv = x_ref[...]
    t0 = t_ref[0:1, :]
    t1 = t_ref[1:2, :]
    emb = jnp.where(r >= 94, t1, t0)
    o_ref[...] = jnp.where(m != 0, xv + emb, xv)


def kernel(x, resids, mask, table):
    n, d = x.shape
    b = _BLOCK
    grid = (n // b,)
    r2 = resids.astype(jnp.int32).reshape(n, 1)
    m2 = mask.astype(jnp.int32).reshape(n, 1)
    return pl.pallas_call(
        _body,
        grid=grid,
        in_specs=[
            pl.BlockSpec((b, d), lambda i: (i, 0)),
            pl.BlockSpec((b, 1), lambda i: (i, 0)),
            pl.BlockSpec((b, 1), lambda i: (i, 0)),
            pl.BlockSpec((2, d), lambda i: (0, 0)),
        ],
        out_specs=pl.BlockSpec((b, d), lambda i: (i, 0)),
        out_shape=jax.ShapeDtypeStruct((n, d), x.dtype),
    )(x, r2, m2, table)
